# Initial kernel scaffold; baseline (speedup 1.0000x reference)
import jax
import jax.numpy as jnp
from jax.experimental import pallas as pl


def kernel(input_seq, label, mask, prototypes):
    raise NotImplementedError("write your pallas kernel here")


# speedup vs baseline: 5.7760x; 5.7760x over previous
import functools

import jax
import jax.numpy as jnp
from jax import lax
from jax.experimental import pallas as pl
from jax.experimental.pallas import tpu as pltpu
from jax.experimental.pallas import tpu_sc as plsc

_B, _P, _T, _C = 256, 128, 365, 8
_TC_FLAT = _T * _C
_PAD_FLAT = 2944
_NC, _NS = 2, 16
_NW = _NC * _NS
_B_PER_W = _B // _NW


def _tc_body(mask_ref, xT_ref, pT_ref, dist_ref, idx_ref):
    m = mask_ref[...]
    acc = jnp.zeros((_B, _P), jnp.float32)
    a = jnp.zeros((_B,), jnp.float32)
    psq = jnp.zeros((_T, _P), jnp.float32)
    for c in range(_C):
        xc = xT_ref[c]
        pc = pT_ref[c]
        xm = xc * m
        acc = acc + jnp.dot(xm, pc, precision=jax.lax.Precision.HIGHEST,
                            preferred_element_type=jnp.float32)
        a = a + jnp.sum(xm * xc, axis=1)
        psq = psq + pc * pc
    m2 = jnp.dot(m, psq, precision=jax.lax.Precision.HIGHEST,
                 preferred_element_type=jnp.float32)
    dist = a[:, None] - 2.0 * acc + m2
    dist_ref[...] = dist
    dmin = jnp.min(dist, axis=1, keepdims=True)
    lane = lax.broadcasted_iota(jnp.int32, (_B, _P), 1)
    idx = jnp.min(jnp.where(dist == dmin, lane, _P), axis=1)
    idx_ref[...] = idx.astype(jnp.int32)


def _tc_distances(mask, xT, pT):
    return pl.pallas_call(
        _tc_body,
        out_shape=(
            jax.ShapeDtypeStruct((_B, _P), jnp.float32),
            jax.ShapeDtypeStruct((_B,), jnp.int32),
        ),
    )(mask, xT, pT)


def _sc_gather(table, idx):
    mesh = plsc.VectorSubcoreMesh(core_axis_name="c", subcore_axis_name="s")

    @functools.partial(
        pl.kernel,
        mesh=mesh,
        out_type=jax.ShapeDtypeStruct((_B, _PAD_FLAT), jnp.float32),
        scratch_types=[
            pltpu.VMEM((_B_PER_W,), jnp.int32),
            pltpu.VMEM((_B_PER_W, _PAD_FLAT), jnp.float32),
            pltpu.SemaphoreType.DMA,
        ],
    )
    def k(table_hbm, idx_hbm, out_hbm, idx_v, rows_v, sem):
        wid = lax.axis_index("s") * _NC + lax.axis_index("c")
        base = wid * _B_PER_W
        pltpu.sync_copy(idx_hbm.at[pl.ds(base, _B_PER_W)], idx_v)
        pltpu.async_copy(table_hbm.at[idx_v], rows_v, sem).wait()
        pltpu.sync_copy(rows_v, out_hbm.at[pl.ds(base, _B_PER_W)])

    return k(table, idx)


def kernel(input_seq, label, mask, prototypes):
    B, T, C = input_seq.shape
    xT = jnp.transpose(input_seq, (2, 0, 1))
    pT = jnp.transpose(prototypes, (2, 1, 0))
    dist, idx = _tc_distances(mask, xT, pT)
    table = jnp.pad(prototypes.reshape(_P, _TC_FLAT),
                    ((0, 0), (0, _PAD_FLAT - _TC_FLAT)))
    gathered = _sc_gather(table, idx)
    output_seq = gathered[:, :_TC_FLAT].reshape(B, T, C)
    return (output_seq, input_seq, dist, idx, label.reshape(B), mask.reshape(B, T))
